# baseline (device time: 298265 ns/iter reference)
import jax
import jax.numpy as jnp
from jax import lax
from jax.experimental import pallas as pl
from jax.experimental.pallas import tpu as pltpu

N_DEV = 4
B = 2
S = 512
H = 8
D = 64
E = 768
BLK = 64
NG = 4
GROWS = 2 * BLK


def kernel(x, Wq, K_ext, V_ext, Wo):
    def body(x_ref, wq_ref, k_ref, v_ref, wo_ref, out_ref,
             xperm, qperm, kcomm, vcomm, num, den,
             ksend, krecv, vsend, vrecv):
        my = lax.axis_index("i")
        left = lax.rem(my + N_DEV - 1, N_DEV)
        right = lax.rem(my + 1, N_DEV)

        barrier_sem = pltpu.get_barrier_semaphore()
        for nbr in (left, right):
            pl.semaphore_signal(
                barrier_sem, inc=1,
                device_id=(nbr,), device_id_type=pl.DeviceIdType.MESH,
            )
        pl.semaphore_wait(barrier_sem, 2)

        def make_rdma(h):
            ksrc = k_ref if h == 0 else kcomm.at[h - 1]
            vsrc = v_ref if h == 0 else vcomm.at[h - 1]
            rk = pltpu.make_async_remote_copy(
                src_ref=ksrc, dst_ref=kcomm.at[h],
                send_sem=ksend.at[h], recv_sem=krecv.at[h],
                device_id=(right,), device_id_type=pl.DeviceIdType.MESH,
            )
            rv = pltpu.make_async_remote_copy(
                src_ref=vsrc, dst_ref=vcomm.at[h],
                send_sem=vsend.at[h], recv_sem=vrecv.at[h],
                device_id=(right,), device_id_type=pl.DeviceIdType.MESH,
            )
            return rk, rv

        for b in range(B):
            for blk in range(S // BLK):
                r = b * S + (blk % NG) * GROWS + (blk // NG) * BLK
                xperm[r:r + BLK, :] = x_ref[b, blk * BLK:(blk + 1) * BLK, :]
        qperm[...] = jnp.dot(xperm[...], wq_ref[...],
                             preferred_element_type=jnp.float32)

        num[...] = jnp.zeros_like(num)
        den[...] = jnp.zeros_like(den)

        def compute_chunk(kc, vc):
            for b in range(B):
                for g in range(NG):
                    r0 = b * S + g * GROWS
                    lo = slice(g * BLK, (g + 1) * BLK)
                    hi = slice((g + NG) * BLK, (g + NG + 1) * BLK)
                    for h in range(H):
                        kg = jnp.concatenate(
                            [kc[b, lo, h, :], kc[b, hi, h, :]], axis=0)
                        vg = jnp.concatenate(
                            [vc[b, lo, h, :], vc[b, hi, h, :]], axis=0)
                        qg = qperm[r0:r0 + GROWS, h * D:(h + 1) * D]
                        s = lax.dot_general(
                            qg, kg, (((1,), (1,)), ((), ())),
                            preferred_element_type=jnp.float32)
                        w = jnp.exp(s * 0.125)
                        num[r0:r0 + GROWS, h * D:(h + 1) * D] += jnp.dot(
                            w, vg, preferred_element_type=jnp.float32)
                        den[r0:r0 + GROWS, h:h + 1] += jnp.sum(
                            w, axis=1, keepdims=True)

        rk, rv = make_rdma(0)
        rk.start()
        rv.start()
        compute_chunk(k_ref, v_ref)
        rk.wait()
        rv.wait()
        for hop in range(1, N_DEV - 1):
            rk, rv = make_rdma(hop)
            rk.start()
            rv.start()
            compute_chunk(kcomm.at[hop - 1], vcomm.at[hop - 1])
            rk.wait()
            rv.wait()
        compute_chunk(kcomm.at[N_DEV - 2], vcomm.at[N_DEV - 2])

        for b in range(B):
            for h in range(H):
                r0 = b * S
                num[r0:r0 + S, h * D:(h + 1) * D] /= den[r0:r0 + S, h:h + 1]

        ctx = qperm
        for b in range(B):
            for blk in range(S // BLK):
                r = b * S + (blk % NG) * GROWS + (blk // NG) * BLK
                ctx[b * S + blk * BLK:b * S + (blk + 1) * BLK, :] = \
                    num[r:r + BLK, :]

        for b in range(B):
            out_ref[b, :, :] = jnp.dot(
                ctx[b * S:(b + 1) * S, :], wo_ref[...],
                preferred_element_type=jnp.float32)

    out_shape = jax.ShapeDtypeStruct((B, S, E), jnp.float32)
    return pl.pallas_call(
        body,
        out_shape=out_shape,
        in_specs=[pl.BlockSpec(memory_space=pltpu.VMEM)] * 5,
        out_specs=pl.BlockSpec(memory_space=pltpu.VMEM),
        scratch_shapes=[
            pltpu.VMEM((B * S, E), jnp.float32),
            pltpu.VMEM((B * S, H * D), jnp.float32),
            pltpu.VMEM((N_DEV - 1, B, S, H, D), jnp.float32),
            pltpu.VMEM((N_DEV - 1, B, S, H, D), jnp.float32),
            pltpu.VMEM((B * S, H * D), jnp.float32),
            pltpu.VMEM((B * S, H), jnp.float32),
            pltpu.SemaphoreType.DMA((N_DEV - 1,)),
            pltpu.SemaphoreType.DMA((N_DEV - 1,)),
            pltpu.SemaphoreType.DMA((N_DEV - 1,)),
            pltpu.SemaphoreType.DMA((N_DEV - 1,)),
        ],
        compiler_params=pltpu.CompilerParams(collective_id=0),
    )(x, Wq, K_ext, V_ext, Wo)


# device time: 105346 ns/iter; 2.8313x vs baseline; 2.8313x over previous
import jax
import jax.numpy as jnp
from jax import lax
from jax.experimental import pallas as pl
from jax.experimental.pallas import tpu as pltpu

N_DEV = 4
B = 2
S = 512
H = 8
D = 64
E = 768
BLK = 64
NG = 4
GROWS = 2 * BLK


def _gr(b, g):
    return (b * NG + g) * (N_DEV * GROWS)


def kernel(x, Wq, K_ext, V_ext, Wo):
    def body(x_ref, wq_ref, k_ref, v_ref, wo_ref, out_ref,
             xperm, qperm, kgath, vgath, num, den,
             ksend, krecv, vsend, vrecv):
        my = lax.axis_index("i")
        left = lax.rem(my + N_DEV - 1, N_DEV)
        right = lax.rem(my + 1, N_DEV)

        barrier_sem = pltpu.get_barrier_semaphore()
        for nbr in (left, right):
            pl.semaphore_signal(
                barrier_sem, inc=1,
                device_id=(nbr,), device_id_type=pl.DeviceIdType.MESH,
            )
        pl.semaphore_wait(barrier_sem, 2)

        def hop_rdmas(h):
            rdmas = []
            for b in range(B):
                tgt = right if b == 0 else left
                for g in range(NG):
                    if h == 0:
                        for s_ in range(2):
                            src_row = b * S + (g + NG * s_) * BLK
                            dst_row = _gr(b, g) + GROWS + s_ * BLK
                            for t, (sref, gref, ss, rs) in enumerate(
                                    ((k_ref, kgath, ksend, krecv),
                                     (v_ref, vgath, vsend, vrecv))):
                                rdmas.append(pltpu.make_async_remote_copy(
                                    src_ref=sref.at[pl.ds(src_row, BLK)],
                                    dst_ref=gref.at[pl.ds(dst_row, BLK)],
                                    send_sem=ss.at[h, b, g * 2 + s_],
                                    recv_sem=rs.at[h, b, g * 2 + s_],
                                    device_id=(tgt,),
                                    device_id_type=pl.DeviceIdType.MESH,
                                ))
                    else:
                        src_row = _gr(b, g) + h * GROWS
                        dst_row = _gr(b, g) + (h + 1) * GROWS
                        for sref, gref, ss, rs in (
                                (kgath, kgath, ksend, krecv),
                                (vgath, vgath, vsend, vrecv)):
                            rdmas.append(pltpu.make_async_remote_copy(
                                src_ref=gref.at[pl.ds(src_row, GROWS)],
                                dst_ref=gref.at[pl.ds(dst_row, GROWS)],
                                send_sem=ss.at[h, b, g],
                                recv_sem=rs.at[h, b, g],
                                device_id=(tgt,),
                                device_id_type=pl.DeviceIdType.MESH,
                            ))
            return rdmas

        h0 = hop_rdmas(0)
        for r in h0:
            r.start()

        for b in range(B):
            for g in range(NG):
                for s_ in range(2):
                    src_row = b * S + (g + NG * s_) * BLK
                    dst_row = _gr(b, g) + s_ * BLK
                    kgath[dst_row:dst_row + BLK, :] = \
                        k_ref[src_row:src_row + BLK, :]
                    vgath[dst_row:dst_row + BLK, :] = \
                        v_ref[src_row:src_row + BLK, :]

        for b in range(B):
            for blk in range(S // BLK):
                r = b * S + (blk % NG) * GROWS + (blk // NG) * BLK
                xperm[r:r + BLK, :] = x_ref[b * S + blk * BLK:
                                            b * S + (blk + 1) * BLK, :]
        qperm[...] = jnp.dot(xperm[...], wq_ref[...],
                             preferred_element_type=jnp.float32)

        def attn_stage(b, c0, nc, first):
            rows = nc * GROWS
            for g in range(NG):
                q0 = b * S + g * GROWS
                kv0 = _gr(b, g) + c0 * GROWS
                for h in range(H):
                    qg = qperm[q0:q0 + GROWS, h * D:(h + 1) * D]
                    kk = kgath[kv0:kv0 + rows, h * D:(h + 1) * D]
                    vv = vgath[kv0:kv0 + rows, h * D:(h + 1) * D]
                    s = lax.dot_general(
                        qg, kk, (((1,), (1,)), ((), ())),
                        preferred_element_type=jnp.float32)
                    w = jnp.exp(s * 0.125)
                    pv = jnp.dot(w, vv, preferred_element_type=jnp.float32)
                    ds_ = jnp.sum(w, axis=1, keepdims=True)
                    if first:
                        num[q0:q0 + GROWS, h * D:(h + 1) * D] = pv
                        den[q0:q0 + GROWS, h:h + 1] = ds_
                    else:
                        num[q0:q0 + GROWS, h * D:(h + 1) * D] += pv
                        den[q0:q0 + GROWS, h:h + 1] += ds_

        for r in h0:
            r.wait()
        h1 = hop_rdmas(1)
        for r in h1:
            r.start()
        attn_stage(0, 0, 2, True)
        for r in h1:
            r.wait()
        h2 = hop_rdmas(2)
        for r in h2:
            r.start()
        attn_stage(1, 0, 2, True)
        for r in h2:
            r.wait()
        attn_stage(0, 2, 2, False)
        attn_stage(1, 2, 2, False)

        for b in range(B):
            for h in range(H):
                r0 = b * S
                num[r0:r0 + S, h * D:(h + 1) * D] /= den[r0:r0 + S, h:h + 1]

        ctx = qperm
        for b in range(B):
            for blk in range(S // BLK):
                r = b * S + (blk % NG) * GROWS + (blk // NG) * BLK
                ctx[b * S + blk * BLK:b * S + (blk + 1) * BLK, :] = \
                    num[r:r + BLK, :]

        for b in range(B):
            out_ref[b, :, :] = jnp.dot(
                ctx[b * S:(b + 1) * S, :], wo_ref[...],
                preferred_element_type=jnp.float32)

    x2 = x.reshape(B * S, E)
    k2 = K_ext.reshape(B * S, H * D)
    v2 = V_ext.reshape(B * S, H * D)
    out_shape = jax.ShapeDtypeStruct((B, S, E), jnp.float32)
    return pl.pallas_call(
        body,
        out_shape=out_shape,
        in_specs=[pl.BlockSpec(memory_space=pltpu.VMEM)] * 5,
        out_specs=pl.BlockSpec(memory_space=pltpu.VMEM),
        scratch_shapes=[
            pltpu.VMEM((B * S, E), jnp.float32),
            pltpu.VMEM((B * S, H * D), jnp.float32),
            pltpu.VMEM((B * NG * N_DEV * GROWS, H * D), jnp.float32),
            pltpu.VMEM((B * NG * N_DEV * GROWS, H * D), jnp.float32),
            pltpu.VMEM((B * S, H * D), jnp.float32),
            pltpu.VMEM((B * S, H), jnp.float32),
            pltpu.SemaphoreType.DMA((N_DEV - 1, B, 2 * NG)),
            pltpu.SemaphoreType.DMA((N_DEV - 1, B, 2 * NG)),
            pltpu.SemaphoreType.DMA((N_DEV - 1, B, 2 * NG)),
            pltpu.SemaphoreType.DMA((N_DEV - 1, B, 2 * NG)),
        ],
        compiler_params=pltpu.CompilerParams(collective_id=0),
    )(x2, Wq, k2, v2, Wo)


# device time: 71077 ns/iter; 4.1964x vs baseline; 1.4821x over previous
import jax
import jax.numpy as jnp
from jax import lax
from jax.experimental import pallas as pl
from jax.experimental.pallas import tpu as pltpu

N_DEV = 4
B = 2
S = 512
H = 8
D = 64
E = 768
BLK = 64
NG = 4
GROWS = 2 * BLK


def _gr(b, g):
    return (b * NG + g) * (N_DEV * GROWS)


def kernel(x, Wq, K_ext, V_ext, Wo):
    def body(x_ref, wq_ref, k_ref, v_ref, wo_ref, out_ref,
             xperm, qperm, qb16, kb16, vb16, kgath, vgath, num, den,
             ksend, krecv, vsend, vrecv):
        my = lax.axis_index("i")
        left = lax.rem(my + N_DEV - 1, N_DEV)
        right = lax.rem(my + 1, N_DEV)

        kb16[...] = k_ref[...].astype(jnp.bfloat16)
        vb16[...] = v_ref[...].astype(jnp.bfloat16)

        barrier_sem = pltpu.get_barrier_semaphore()
        for nbr in (left, right):
            pl.semaphore_signal(
                barrier_sem, inc=1,
                device_id=(nbr,), device_id_type=pl.DeviceIdType.MESH,
            )
        pl.semaphore_wait(barrier_sem, 2)

        def hop_rdmas(h):
            rdmas = []
            for b in range(B):
                tgt = right if b == 0 else left
                for g in range(NG):
                    if h == 0:
                        for s_ in range(2):
                            src_row = b * S + (g + NG * s_) * BLK
                            dst_row = _gr(b, g) + GROWS + s_ * BLK
                            for sref, gref, ss, rs in (
                                    (kb16, kgath, ksend, krecv),
                                    (vb16, vgath, vsend, vrecv)):
                                rdmas.append(pltpu.make_async_remote_copy(
                                    src_ref=sref.at[pl.ds(src_row, BLK)],
                                    dst_ref=gref.at[pl.ds(dst_row, BLK)],
                                    send_sem=ss.at[h, b, g * 2 + s_],
                                    recv_sem=rs.at[h, b, g * 2 + s_],
                                    device_id=(tgt,),
                                    device_id_type=pl.DeviceIdType.MESH,
                                ))
                    else:
                        src_row = _gr(b, g) + h * GROWS
                        dst_row = _gr(b, g) + (h + 1) * GROWS
                        for gref, ss, rs in (
                                (kgath, ksend, krecv),
                                (vgath, vsend, vrecv)):
                            rdmas.append(pltpu.make_async_remote_copy(
                                src_ref=gref.at[pl.ds(src_row, GROWS)],
                                dst_ref=gref.at[pl.ds(dst_row, GROWS)],
                                send_sem=ss.at[h, b, g],
                                recv_sem=rs.at[h, b, g],
                                device_id=(tgt,),
                                device_id_type=pl.DeviceIdType.MESH,
                            ))
            return rdmas

        h0 = hop_rdmas(0)
        for r in h0:
            r.start()

        for b in range(B):
            for g in range(NG):
                for s_ in range(2):
                    src_row = b * S + (g + NG * s_) * BLK
                    dst_row = _gr(b, g) + s_ * BLK
                    kgath[dst_row:dst_row + BLK, :] = \
                        kb16[src_row:src_row + BLK, :]
                    vgath[dst_row:dst_row + BLK, :] = \
                        vb16[src_row:src_row + BLK, :]

        for b in range(B):
            for blk in range(S // BLK):
                r = b * S + (blk % NG) * GROWS + (blk // NG) * BLK
                xperm[r:r + BLK, :] = x_ref[b * S + blk * BLK:
                                            b * S + (blk + 1) * BLK, :]
        qperm[...] = jnp.dot(xperm[...], wq_ref[...],
                             preferred_element_type=jnp.float32)
        qb16[...] = qperm[...].astype(jnp.bfloat16)

        def attn_stage(b, c0, nc, first):
            rows = nc * GROWS
            for g in range(NG):
                q0 = b * S + g * GROWS
                kv0 = _gr(b, g) + c0 * GROWS
                for h in range(H):
                    qg = qb16[q0:q0 + GROWS, h * D:(h + 1) * D]
                    kk = kgath[kv0:kv0 + rows, h * D:(h + 1) * D]
                    vv = vgath[kv0:kv0 + rows, h * D:(h + 1) * D]
                    s = lax.dot_general(
                        qg, kk, (((1,), (1,)), ((), ())),
                        preferred_element_type=jnp.float32)
                    w = jnp.exp(s * 0.125)
                    wb = w.astype(jnp.bfloat16)
                    pv = jnp.dot(wb, vv, preferred_element_type=jnp.float32)
                    ds_ = jnp.sum(w, axis=1, keepdims=True)
                    if first:
                        num[q0:q0 + GROWS, h * D:(h + 1) * D] = pv
                        den[q0:q0 + GROWS, h:h + 1] = ds_
                    else:
                        num[q0:q0 + GROWS, h * D:(h + 1) * D] += pv
                        den[q0:q0 + GROWS, h:h + 1] += ds_

        for r in h0:
            r.wait()
        h1 = hop_rdmas(1)
        for r in h1:
            r.start()
        attn_stage(0, 0, 2, True)
        for r in h1:
            r.wait()
        h2 = hop_rdmas(2)
        for r in h2:
            r.start()
        attn_stage(1, 0, 2, True)
        for r in h2:
            r.wait()
        attn_stage(0, 2, 2, False)
        attn_stage(1, 2, 2, False)

        for b in range(B):
            for h in range(H):
                r0 = b * S
                num[r0:r0 + S, h * D:(h + 1) * D] /= den[r0:r0 + S, h:h + 1]

        ctx = qperm
        for b in range(B):
            for blk in range(S // BLK):
                r = b * S + (blk % NG) * GROWS + (blk // NG) * BLK
                ctx[b * S + blk * BLK:b * S + (blk + 1) * BLK, :] = \
                    num[r:r + BLK, :]

        for b in range(B):
            out_ref[b, :, :] = jnp.dot(
                ctx[b * S:(b + 1) * S, :], wo_ref[...],
                preferred_element_type=jnp.float32)

    x2 = x.reshape(B * S, E)
    k2 = K_ext.reshape(B * S, H * D)
    v2 = V_ext.reshape(B * S, H * D)
    out_shape = jax.ShapeDtypeStruct((B, S, E), jnp.float32)
    return pl.pallas_call(
        body,
        out_shape=out_shape,
        in_specs=[pl.BlockSpec(memory_space=pltpu.VMEM)] * 5,
        out_specs=pl.BlockSpec(memory_space=pltpu.VMEM),
        scratch_shapes=[
            pltpu.VMEM((B * S, E), jnp.float32),
            pltpu.VMEM((B * S, H * D), jnp.float32),
            pltpu.VMEM((B * S, H * D), jnp.bfloat16),
            pltpu.VMEM((B * S, H * D), jnp.bfloat16),
            pltpu.VMEM((B * S, H * D), jnp.bfloat16),
            pltpu.VMEM((B * NG * N_DEV * GROWS, H * D), jnp.bfloat16),
            pltpu.VMEM((B * NG * N_DEV * GROWS, H * D), jnp.bfloat16),
            pltpu.VMEM((B * S, H * D), jnp.float32),
            pltpu.VMEM((B * S, H), jnp.float32),
            pltpu.SemaphoreType.DMA((N_DEV - 1, B, 2 * NG)),
            pltpu.SemaphoreType.DMA((N_DEV - 1, B, 2 * NG)),
            pltpu.SemaphoreType.DMA((N_DEV - 1, B, 2 * NG)),
            pltpu.SemaphoreType.DMA((N_DEV - 1, B, 2 * NG)),
        ],
        compiler_params=pltpu.CompilerParams(collective_id=0),
    )(x2, Wq, k2, v2, Wo)


# device time: 66721 ns/iter; 4.4703x vs baseline; 1.0653x over previous
import jax
import jax.numpy as jnp
from jax import lax
from jax.experimental import pallas as pl
from jax.experimental.pallas import tpu as pltpu

N_DEV = 4
B = 2
S = 512
H = 8
D = 64
E = 768
BLK = 64
NG = 4
GROWS = 2 * BLK


def _gr(b, g):
    return (b * NG + g) * (N_DEV * GROWS)


def kernel(x, Wq, K_ext, V_ext, Wo):
    def body(x_ref, wq_ref, k_ref, v_ref, wo_ref, out_ref,
             xb16, wqb16, wob16, qb16, kb16, vb16, kgath, vgath, num, den,
             ksend, krecv, vsend, vrecv):
        my = lax.axis_index("i")
        left = lax.rem(my + N_DEV - 1, N_DEV)
        right = lax.rem(my + 1, N_DEV)

        def hop_rdmas(h):
            krd, vrd = [], []
            for b in range(B):
                tgt = right if b == 0 else left
                for g in range(NG):
                    if h == 0:
                        for s_ in range(2):
                            src_row = b * S + (g + NG * s_) * BLK
                            dst_row = _gr(b, g) + GROWS + s_ * BLK
                            for acc, sref, gref, ss, rs in (
                                    (krd, kb16, kgath, ksend, krecv),
                                    (vrd, vb16, vgath, vsend, vrecv)):
                                acc.append(pltpu.make_async_remote_copy(
                                    src_ref=sref.at[pl.ds(src_row, BLK)],
                                    dst_ref=gref.at[pl.ds(dst_row, BLK)],
                                    send_sem=ss.at[h, b, g * 2 + s_],
                                    recv_sem=rs.at[h, b, g * 2 + s_],
                                    device_id=(tgt,),
                                    device_id_type=pl.DeviceIdType.MESH,
                                ))
                    else:
                        src_row = _gr(b, g) + h * GROWS
                        dst_row = _gr(b, g) + (h + 1) * GROWS
                        for acc, gref, ss, rs in (
                                (krd, kgath, ksend, krecv),
                                (vrd, vgath, vsend, vrecv)):
                            acc.append(pltpu.make_async_remote_copy(
                                src_ref=gref.at[pl.ds(src_row, GROWS)],
                                dst_ref=gref.at[pl.ds(dst_row, GROWS)],
                                send_sem=ss.at[h, b, g],
                                recv_sem=rs.at[h, b, g],
                                device_id=(tgt,),
                                device_id_type=pl.DeviceIdType.MESH,
                            ))
            return krd, vrd

        kb16[...] = k_ref[...].astype(jnp.bfloat16)

        barrier_sem = pltpu.get_barrier_semaphore()
        for nbr in (left, right):
            pl.semaphore_signal(
                barrier_sem, inc=1,
                device_id=(nbr,), device_id_type=pl.DeviceIdType.MESH,
            )
        pl.semaphore_wait(barrier_sem, 2)

        k0, v0 = hop_rdmas(0)
        for r in k0:
            r.start()
        vb16[...] = v_ref[...].astype(jnp.bfloat16)
        for r in v0:
            r.start()

        for b in range(B):
            for g in range(NG):
                for s_ in range(2):
                    src_row = b * S + (g + NG * s_) * BLK
                    dst_row = _gr(b, g) + s_ * BLK
                    kgath[dst_row:dst_row + BLK, :] = \
                        kb16[src_row:src_row + BLK, :]
                    vgath[dst_row:dst_row + BLK, :] = \
                        vb16[src_row:src_row + BLK, :]

        for b in range(B):
            for blk in range(S // BLK):
                r = b * S + (blk % NG) * GROWS + (blk // NG) * BLK
                xb16[r:r + BLK, :] = x_ref[b * S + blk * BLK:
                                           b * S + (blk + 1) * BLK,
                                           :].astype(jnp.bfloat16)
        wqb16[...] = wq_ref[...].astype(jnp.bfloat16)
        wob16[...] = wo_ref[...].astype(jnp.bfloat16)
        qb16[...] = jnp.dot(xb16[...], wqb16[...],
                            preferred_element_type=jnp.float32
                            ).astype(jnp.bfloat16)

        def attn_stage(b, c0, nc, first):
            rows = nc * GROWS
            for g in range(NG):
                q0 = b * S + g * GROWS
                kv0 = _gr(b, g) + c0 * GROWS
                for h in range(H):
                    qg = qb16[q0:q0 + GROWS, h * D:(h + 1) * D]
                    kk = kgath[kv0:kv0 + rows, h * D:(h + 1) * D]
                    vv = vgath[kv0:kv0 + rows, h * D:(h + 1) * D]
                    s = lax.dot_general(
                        qg, kk, (((1,), (1,)), ((), ())),
                        preferred_element_type=jnp.float32)
                    w = jnp.exp(s * 0.125)
                    wb = w.astype(jnp.bfloat16)
                    pv = jnp.dot(wb, vv, preferred_element_type=jnp.float32)
                    ds_ = jnp.sum(w, axis=1, keepdims=True)
                    if first:
                        num[q0:q0 + GROWS, h * D:(h + 1) * D] = pv
                        den[q0:q0 + GROWS, h:h + 1] = ds_
                    else:
                        num[q0:q0 + GROWS, h * D:(h + 1) * D] += pv
                        den[q0:q0 + GROWS, h:h + 1] += ds_

        for r in k0 + v0:
            r.wait()
        k1, v1 = hop_rdmas(1)
        for r in k1 + v1:
            r.start()
        attn_stage(0, 0, 2, True)
        attn_stage(1, 0, 2, True)
        for r in k1 + v1:
            r.wait()
        k2, v2_ = hop_rdmas(2)
        for r in k2 + v2_:
            r.start()
        attn_stage(0, 2, 1, False)
        attn_stage(1, 2, 1, False)
        for r in k2 + v2_:
            r.wait()
        attn_stage(0, 3, 1, False)
        attn_stage(1, 3, 1, False)

        den[...] = 1.0 / den[...]
        for b in range(B):
            for h in range(H):
                r0 = b * S
                num[r0:r0 + S, h * D:(h + 1) * D] *= den[r0:r0 + S, h:h + 1]

        for b in range(B):
            for blk in range(S // BLK):
                r = b * S + (blk % NG) * GROWS + (blk // NG) * BLK
                qb16[b * S + blk * BLK:b * S + (blk + 1) * BLK, :] = \
                    num[r:r + BLK, :].astype(jnp.bfloat16)

        for b in range(B):
            out_ref[b, :, :] = jnp.dot(
                qb16[b * S:(b + 1) * S, :], wob16[...],
                preferred_element_type=jnp.float32)

    x2 = x.reshape(B * S, E)
    k2 = K_ext.reshape(B * S, H * D)
    v2 = V_ext.reshape(B * S, H * D)
    out_shape = jax.ShapeDtypeStruct((B, S, E), jnp.float32)
    return pl.pallas_call(
        body,
        out_shape=out_shape,
        in_specs=[pl.BlockSpec(memory_space=pltpu.VMEM)] * 5,
        out_specs=pl.BlockSpec(memory_space=pltpu.VMEM),
        scratch_shapes=[
            pltpu.VMEM((B * S, E), jnp.bfloat16),
            pltpu.VMEM((E, H * D), jnp.bfloat16),
            pltpu.VMEM((H * D, E), jnp.bfloat16),
            pltpu.VMEM((B * S, H * D), jnp.bfloat16),
            pltpu.VMEM((B * S, H * D), jnp.bfloat16),
            pltpu.VMEM((B * S, H * D), jnp.bfloat16),
            pltpu.VMEM((B * NG * N_DEV * GROWS, H * D), jnp.bfloat16),
            pltpu.VMEM((B * NG * N_DEV * GROWS, H * D), jnp.bfloat16),
            pltpu.VMEM((B * S, H * D), jnp.float32),
            pltpu.VMEM((B * S, H), jnp.float32),
            pltpu.SemaphoreType.DMA((N_DEV - 1, B, 2 * NG)),
            pltpu.SemaphoreType.DMA((N_DEV - 1, B, 2 * NG)),
            pltpu.SemaphoreType.DMA((N_DEV - 1, B, 2 * NG)),
            pltpu.SemaphoreType.DMA((N_DEV - 1, B, 2 * NG)),
        ],
        compiler_params=pltpu.CompilerParams(collective_id=0),
    )(x2, Wq, k2, v2, Wo)


# device time: 58504 ns/iter; 5.0982x vs baseline; 1.1405x over previous
import jax
import jax.numpy as jnp
from jax import lax
from jax.experimental import pallas as pl
from jax.experimental.pallas import tpu as pltpu

N_DEV = 4
B = 2
S = 512
H = 8
D = 64
E = 768
BLK = 64
NG = 4
GROWS = 2 * BLK

F8 = jnp.float8_e4m3fn


def _gr(b, g):
    return (b * NG + g) * (N_DEV * GROWS)


def kernel(x, Wq, K_ext, V_ext, Wo):
    def body(x_ref, wq_ref, k_ref, v_ref, wo_ref, out_ref,
             xb16, wqb16, wob16, qb16, kb8, vb8, kwire, vwire,
             kgath, vgath, num, den,
             ksend, krecv, vsend, vrecv):
        my = lax.axis_index("i")
        left = lax.rem(my + N_DEV - 1, N_DEV)
        right = lax.rem(my + 1, N_DEV)

        def hop_rdmas(h):
            krd, vrd = [], []
            for b in range(B):
                tgt = right if b == 0 else left
                for g in range(NG):
                    if h == 0:
                        for s_ in range(2):
                            src_row = b * S + (g + NG * s_) * BLK
                            dst_row = _gr(b, g) + GROWS + s_ * BLK
                            for acc, sref, wref, ss, rs in (
                                    (krd, kb8, kwire, ksend, krecv),
                                    (vrd, vb8, vwire, vsend, vrecv)):
                                acc.append(pltpu.make_async_remote_copy(
                                    src_ref=sref.at[pl.ds(src_row, BLK)],
                                    dst_ref=wref.at[pl.ds(dst_row, BLK)],
                                    send_sem=ss.at[h, b, g * 2 + s_],
                                    recv_sem=rs.at[h, b, g * 2 + s_],
                                    device_id=(tgt,),
                                    device_id_type=pl.DeviceIdType.MESH,
                                ))
                    else:
                        src_row = _gr(b, g) + h * GROWS
                        dst_row = _gr(b, g) + (h + 1) * GROWS
                        for acc, wref, ss, rs in (
                                (krd, kwire, ksend, krecv),
                                (vrd, vwire, vsend, vrecv)):
                            acc.append(pltpu.make_async_remote_copy(
                                src_ref=wref.at[pl.ds(src_row, GROWS)],
                                dst_ref=wref.at[pl.ds(dst_row, GROWS)],
                                send_sem=ss.at[h, b, g],
                                recv_sem=rs.at[h, b, g],
                                device_id=(tgt,),
                                device_id_type=pl.DeviceIdType.MESH,
                            ))
            return krd, vrd

        kb8[...] = k_ref[...].astype(F8)

        barrier_sem = pltpu.get_barrier_semaphore()
        for nbr in (left, right):
            pl.semaphore_signal(
                barrier_sem, inc=1,
                device_id=(nbr,), device_id_type=pl.DeviceIdType.MESH,
            )
        pl.semaphore_wait(barrier_sem, 2)

        k0, v0 = hop_rdmas(0)
        for r in k0:
            r.start()
        vb8[...] = v_ref[...].astype(F8)
        for r in v0:
            r.start()

        for b in range(B):
            for g in range(NG):
                for s_ in range(2):
                    src_row = b * S + (g + NG * s_) * BLK
                    dst_row = _gr(b, g) + s_ * BLK
                    kgath[dst_row:dst_row + BLK, :] = \
                        k_ref[src_row:src_row + BLK, :].astype(jnp.bfloat16)
                    vgath[dst_row:dst_row + BLK, :] = \
                        v_ref[src_row:src_row + BLK, :].astype(jnp.bfloat16)

        for b in range(B):
            for blk in range(S // BLK):
                r = b * S + (blk % NG) * GROWS + (blk // NG) * BLK
                xb16[r:r + BLK, :] = x_ref[b * S + blk * BLK:
                                           b * S + (blk + 1) * BLK,
                                           :].astype(jnp.bfloat16)
        wqb16[...] = wq_ref[...].astype(jnp.bfloat16)
        wob16[...] = wo_ref[...].astype(jnp.bfloat16)
        qb16[...] = jnp.dot(xb16[...], wqb16[...],
                            preferred_element_type=jnp.float32
                            ).astype(jnp.bfloat16)

        def upcast_arrivals(c):
            for b in range(B):
                for g in range(NG):
                    r0 = _gr(b, g) + c * GROWS
                    kgath[r0:r0 + GROWS, :] = \
                        kwire[r0:r0 + GROWS, :].astype(jnp.bfloat16)
                    vgath[r0:r0 + GROWS, :] = \
                        vwire[r0:r0 + GROWS, :].astype(jnp.bfloat16)

        def attn_stage(b, c0, nc, first):
            rows = nc * GROWS
            for g in range(NG):
                q0 = b * S + g * GROWS
                kv0 = _gr(b, g) + c0 * GROWS
                for h in range(H):
                    qg = qb16[q0:q0 + GROWS, h * D:(h + 1) * D]
                    kk = kgath[kv0:kv0 + rows, h * D:(h + 1) * D]
                    vv = vgath[kv0:kv0 + rows, h * D:(h + 1) * D]
                    s = lax.dot_general(
                        qg, kk, (((1,), (1,)), ((), ())),
                        preferred_element_type=jnp.float32)
                    w = jnp.exp(s * 0.125)
                    wb = w.astype(jnp.bfloat16)
                    pv = jnp.dot(wb, vv, preferred_element_type=jnp.float32)
                    ds_ = jnp.sum(w, axis=1, keepdims=True)
                    if first:
                        num[q0:q0 + GROWS, h * D:(h + 1) * D] = pv
                        den[q0:q0 + GROWS, h:h + 1] = ds_
                    else:
                        num[q0:q0 + GROWS, h * D:(h + 1) * D] += pv
                        den[q0:q0 + GROWS, h:h + 1] += ds_

        for r in k0 + v0:
            r.wait()
        k1, v1 = hop_rdmas(1)
        for r in k1 + v1:
            r.start()
        upcast_arrivals(1)
        attn_stage(0, 0, 2, True)
        attn_stage(1, 0, 2, True)
        for r in k1 + v1:
            r.wait()
        k2, v2_ = hop_rdmas(2)
        for r in k2 + v2_:
            r.start()
        upcast_arrivals(2)
        attn_stage(0, 2, 1, False)
        attn_stage(1, 2, 1, False)
        for r in k2 + v2_:
            r.wait()
        upcast_arrivals(3)
        attn_stage(0, 3, 1, False)
        attn_stage(1, 3, 1, False)

        den[...] = 1.0 / den[...]
        for b in range(B):
            for h in range(H):
                r0 = b * S
                num[r0:r0 + S, h * D:(h + 1) * D] *= den[r0:r0 + S, h:h + 1]

        for b in range(B):
            for blk in range(S // BLK):
                r = b * S + (blk % NG) * GROWS + (blk // NG) * BLK
                qb16[b * S + blk * BLK:b * S + (blk + 1) * BLK, :] = \
                    num[r:r + BLK, :].astype(jnp.bfloat16)

        for b in range(B):
            out_ref[b, :, :] = jnp.dot(
                qb16[b * S:(b + 1) * S, :], wob16[...],
                preferred_element_type=jnp.float32)

    x2 = x.reshape(B * S, E)
    k2 = K_ext.reshape(B * S, H * D)
    v2 = V_ext.reshape(B * S, H * D)
    out_shape = jax.ShapeDtypeStruct((B, S, E), jnp.float32)
    return pl.pallas_call(
        body,
        out_shape=out_shape,
        in_specs=[pl.BlockSpec(memory_space=pltpu.VMEM)] * 5,
        out_specs=pl.BlockSpec(memory_space=pltpu.VMEM),
        scratch_shapes=[
            pltpu.VMEM((B * S, E), jnp.bfloat16),
            pltpu.VMEM((E, H * D), jnp.bfloat16),
            pltpu.VMEM((H * D, E), jnp.bfloat16),
            pltpu.VMEM((B * S, H * D), jnp.bfloat16),
            pltpu.VMEM((B * S, H * D), F8),
            pltpu.VMEM((B * S, H * D), F8),
            pltpu.VMEM((B * NG * N_DEV * GROWS, H * D), F8),
            pltpu.VMEM((B * NG * N_DEV * GROWS, H * D), F8),
            pltpu.VMEM((B * NG * N_DEV * GROWS, H * D), jnp.bfloat16),
            pltpu.VMEM((B * NG * N_DEV * GROWS, H * D), jnp.bfloat16),
            pltpu.VMEM((B * S, H * D), jnp.float32),
            pltpu.VMEM((B * S, H), jnp.float32),
            pltpu.SemaphoreType.DMA((N_DEV - 1, B, 2 * NG)),
            pltpu.SemaphoreType.DMA((N_DEV - 1, B, 2 * NG)),
            pltpu.SemaphoreType.DMA((N_DEV - 1, B, 2 * NG)),
            pltpu.SemaphoreType.DMA((N_DEV - 1, B, 2 * NG)),
        ],
        compiler_params=pltpu.CompilerParams(collective_id=0),
    )(x2, Wq, k2, v2, Wo)
